# baseline (speedup 1.0000x reference)
import functools

import jax
import jax.numpy as jnp
from jax.experimental import pallas as pl
from jax.experimental.pallas import tpu as pltpu
from jax.sharding import Mesh, PartitionSpec as P

HIDDEN = 2048
NUM_EXPERTS = 16
TOP_K = 2
TM = 1024


def _router_kernel(x_ref, w1_ref, b1_ref, w2t_ref, b2t_ref,
                   idx_ref, probs_ref, psum_ref):
    h = jnp.dot(x_ref[...].astype(jnp.bfloat16), w1_ref[...],
                preferred_element_type=jnp.float32)
    h = jnp.maximum(h + b1_ref[...], 0.0)
    lt = jax.lax.dot_general(w2t_ref[...], h.astype(jnp.bfloat16),
                             (((1,), (1,)), ((), ())),
                             preferred_element_type=jnp.float32)
    lt = lt + b2t_ref[...]

    m1 = jnp.max(lt, axis=0, keepdims=True)
    e = jnp.exp(lt - m1)
    z = jnp.sum(e, axis=0, keepdims=True)
    p = e / z

    psum_ref[...] = jnp.sum(p, axis=1).reshape(1, 1, NUM_EXPERTS)

    sub = jax.lax.broadcasted_iota(jnp.int32, lt.shape, 0)
    i1 = jnp.min(jnp.where(lt == m1, sub, NUM_EXPERTS),
                 axis=0, keepdims=True)
    masked = jnp.where(sub == i1, -jnp.inf, lt)
    m2 = jnp.max(masked, axis=0, keepdims=True)
    i2 = jnp.min(jnp.where(masked == m2, sub, NUM_EXPERTS),
                 axis=0, keepdims=True)

    p1 = jnp.sum(jnp.where(sub == i1, p, 0.0), axis=0, keepdims=True)
    p2 = jnp.sum(jnp.where(sub == i2, p, 0.0), axis=0, keepdims=True)
    s = p1 + p2
    idx_ref[...] = jnp.concatenate([i1, i2], axis=0).T
    probs_ref[...] = jnp.concatenate([p1 / s, p2 / s], axis=0).T


def _aux_kernel(psum_ref, aux_ref, *, total):
    mean = jnp.sum(psum_ref[...], axis=0) / jnp.float32(total)
    aux_ref[...] = jnp.sum(mean * jnp.log(mean * NUM_EXPERTS + 1e-9)
                           ).reshape(1, 1)


def _shard_fn(x2, w1, b1r, w2t, b2t, *, total_tokens):
    m_local, H = x2.shape
    nsteps = m_local // TM

    idx, probs, psums = pl.pallas_call(
        _router_kernel,
        grid=(nsteps,),
        in_specs=[
            pl.BlockSpec((TM, H), lambda i: (i, 0)),
            pl.BlockSpec((H, H), lambda i: (0, 0)),
            pl.BlockSpec((1, H), lambda i: (0, 0)),
            pl.BlockSpec((NUM_EXPERTS, H), lambda i: (0, 0)),
            pl.BlockSpec((NUM_EXPERTS, 1), lambda i: (0, 0)),
        ],
        out_specs=[
            pl.BlockSpec((TM, TOP_K), lambda i: (i, 0)),
            pl.BlockSpec((TM, TOP_K), lambda i: (i, 0)),
            pl.BlockSpec((1, 1, NUM_EXPERTS), lambda i: (i, 0, 0)),
        ],
        out_shape=[
            jax.ShapeDtypeStruct((m_local, TOP_K), jnp.int32),
            jax.ShapeDtypeStruct((m_local, TOP_K), jnp.float32),
            jax.ShapeDtypeStruct((nsteps, 1, NUM_EXPERTS), jnp.float32),
        ],
        compiler_params=pltpu.CompilerParams(
            dimension_semantics=("parallel",),
        ),
    )(x2, w1, b1r, w2t, b2t)

    psums = jax.lax.psum(psums, "d")
    aux = pl.pallas_call(
        functools.partial(_aux_kernel, total=total_tokens),
        out_shape=jax.ShapeDtypeStruct((1, 1), jnp.float32),
    )(psums.reshape(nsteps, NUM_EXPERTS))
    return idx, probs, aux


def kernel(x, W1, b1, W2, b2):
    B, S, H = x.shape
    M = B * S

    devs = jax.devices()
    ndev = len(devs)
    while ndev > 1 and (M % (ndev * TM) != 0):
        ndev -= 1
    mesh = Mesh(devs[:ndev], ("d",))

    x2 = x.reshape(M, H)
    w1 = W1.astype(jnp.bfloat16)
    w2t = W2.T.astype(jnp.bfloat16)
    b1r = b1.reshape(1, H)
    b2t = b2.reshape(NUM_EXPERTS, 1)

    f = jax.shard_map(
        functools.partial(_shard_fn, total_tokens=M),
        mesh=mesh,
        in_specs=(P("d", None), P(None, None), P(None, None),
                  P(None, None), P(None, None)),
        out_specs=(P("d", None), P("d", None), P(None, None)),
        check_vma=False,
    )
    idx, probs, aux = f(x2, w1, b1r, w2t, b2t)

    return (idx.reshape(B, S, TOP_K), probs.reshape(B, S, TOP_K),
            aux.reshape(()))


# speedup vs baseline: 5.1129x; 5.1129x over previous
import functools

import jax
import jax.numpy as jnp
from jax.experimental import pallas as pl
from jax.experimental.pallas import tpu as pltpu

HIDDEN = 2048
NUM_EXPERTS = 16
TOP_K = 2
TM = 1024


def _router_kernel(x_ref, w1_ref, b1_ref, w2t_ref, b2t_ref,
                   idx_ref, probs_ref, psum_ref, w1bf_ref):
    i = pl.program_id(0)

    @pl.when(i == 0)
    def _cast_w1():
        w1bf_ref[...] = w1_ref[...].astype(jnp.bfloat16)

    h = jnp.dot(x_ref[...].astype(jnp.bfloat16), w1bf_ref[...],
                preferred_element_type=jnp.float32)
    h = jnp.maximum(h + b1_ref[...], 0.0)
    lt = jax.lax.dot_general(w2t_ref[...], h.astype(jnp.bfloat16),
                             (((1,), (1,)), ((), ())),
                             preferred_element_type=jnp.float32)
    lt = lt + b2t_ref[...]

    m1 = jnp.max(lt, axis=0, keepdims=True)
    e = jnp.exp(lt - m1)
    z = jnp.sum(e, axis=0, keepdims=True)
    p = e / z

    psum_ref[...] = jnp.sum(p, axis=1).reshape(1, 1, NUM_EXPERTS)

    sub = jax.lax.broadcasted_iota(jnp.int32, lt.shape, 0)
    i1 = jnp.min(jnp.where(lt == m1, sub, NUM_EXPERTS),
                 axis=0, keepdims=True)
    masked = jnp.where(sub == i1, -jnp.inf, lt)
    m2 = jnp.max(masked, axis=0, keepdims=True)
    i2 = jnp.min(jnp.where(masked == m2, sub, NUM_EXPERTS),
                 axis=0, keepdims=True)

    p1 = jnp.sum(jnp.where(sub == i1, p, 0.0), axis=0, keepdims=True)
    p2 = jnp.sum(jnp.where(sub == i2, p, 0.0), axis=0, keepdims=True)
    s = p1 + p2
    idx_ref[...] = jnp.concatenate([i1, i2], axis=0).T
    probs_ref[...] = jnp.concatenate([p1 / s, p2 / s], axis=0).T


def _aux_kernel(psum_ref, aux_ref, *, total):
    mean = jnp.sum(psum_ref[...], axis=0) / jnp.float32(total)
    aux_ref[...] = jnp.sum(mean * jnp.log(mean * NUM_EXPERTS + 1e-9)
                           ).reshape(1, 1)


def kernel(x, W1, b1, W2, b2):
    B, S, H = x.shape
    M = B * S
    x2 = x.reshape(M, H)
    w2t = W2.T.astype(jnp.bfloat16)
    b1r = b1.reshape(1, H)
    b2t = b2.reshape(NUM_EXPERTS, 1)
    nsteps = M // TM

    idx, probs, psums = pl.pallas_call(
        _router_kernel,
        grid=(nsteps,),
        in_specs=[
            pl.BlockSpec((TM, H), lambda i: (i, 0)),
            pl.BlockSpec((H, H), lambda i: (0, 0)),
            pl.BlockSpec((1, H), lambda i: (0, 0)),
            pl.BlockSpec((NUM_EXPERTS, H), lambda i: (0, 0)),
            pl.BlockSpec((NUM_EXPERTS, 1), lambda i: (0, 0)),
        ],
        out_specs=[
            pl.BlockSpec((TM, TOP_K), lambda i: (i, 0)),
            pl.BlockSpec((TM, TOP_K), lambda i: (i, 0)),
            pl.BlockSpec((1, 1, NUM_EXPERTS), lambda i: (i, 0, 0)),
        ],
        out_shape=[
            jax.ShapeDtypeStruct((M, TOP_K), jnp.int32),
            jax.ShapeDtypeStruct((M, TOP_K), jnp.float32),
            jax.ShapeDtypeStruct((nsteps, 1, NUM_EXPERTS), jnp.float32),
        ],
        scratch_shapes=[pltpu.VMEM((HIDDEN, HIDDEN), jnp.bfloat16)],
        compiler_params=pltpu.CompilerParams(
            dimension_semantics=("arbitrary",),
        ),
    )(x2, W1, b1r, w2t, b2t)

    aux = pl.pallas_call(
        functools.partial(_aux_kernel, total=M),
        out_shape=jax.ShapeDtypeStruct((1, 1), jnp.float32),
    )(psums.reshape(nsteps, NUM_EXPERTS))

    return (idx.reshape(B, S, TOP_K), probs.reshape(B, S, TOP_K),
            aux.reshape(()))


# speedup vs baseline: 5.1644x; 1.0101x over previous
import functools

import jax
import jax.numpy as jnp
from jax.experimental import pallas as pl
from jax.experimental.pallas import tpu as pltpu

HIDDEN = 2048
NUM_EXPERTS = 16
TOP_K = 2
TM = 1024


def _top2(lt, p):
    m1 = jnp.max(lt, axis=0, keepdims=True)
    sub = jax.lax.broadcasted_iota(jnp.int32, lt.shape, 0)
    i1 = jnp.min(jnp.where(lt == m1, sub, NUM_EXPERTS), axis=0, keepdims=True)
    masked = jnp.where(sub == i1, -jnp.inf, lt)
    m2 = jnp.max(masked, axis=0, keepdims=True)
    i2 = jnp.min(jnp.where(masked == m2, sub, NUM_EXPERTS),
                 axis=0, keepdims=True)
    p1 = jnp.sum(jnp.where(sub == i1, p, 0.0), axis=0, keepdims=True)
    p2 = jnp.sum(jnp.where(sub == i2, p, 0.0), axis=0, keepdims=True)
    s = p1 + p2
    idx = jnp.concatenate([i1, i2], axis=0).T
    probs = jnp.concatenate([p1 / s, p2 / s], axis=0).T
    return idx, probs


def _softmax_e(lt):
    m1 = jnp.max(lt, axis=0, keepdims=True)
    e = jnp.exp(lt - m1)
    z = jnp.sum(e, axis=0, keepdims=True)
    return e / z


def _router_kernel(x_ref, w1_ref, b1_ref, w2t_ref, b2t_ref,
                   idx_ref, probs_ref, psum_ref, lt_ref, w1bf_ref):
    i = pl.program_id(0)

    @pl.when(i == 0)
    def _cast_w1():
        w1bf_ref[...] = w1_ref[...].astype(jnp.bfloat16)

    lt_prev = lt_ref[...]
    p = _softmax_e(lt_prev)
    psum_ref[...] = jnp.sum(p, axis=1).reshape(1, 1, NUM_EXPERTS)
    idx, probs = _top2(lt_prev, p)
    idx_ref[...] = idx
    probs_ref[...] = probs

    h = jnp.dot(x_ref[...].astype(jnp.bfloat16), w1bf_ref[...],
                preferred_element_type=jnp.float32)
    h = jnp.maximum(h + b1_ref[...], 0.0)
    lt = jax.lax.dot_general(w2t_ref[...], h.astype(jnp.bfloat16),
                             (((1,), (1,)), ((), ())),
                             preferred_element_type=jnp.float32)
    lt_ref[...] = lt + b2t_ref[...]


def _final_kernel(psum_ref, lt_ref, idx_in_ref, probs_in_ref,
                  idx_ref, probs_ref, aux_ref, *, total):
    del idx_in_ref, probs_in_ref
    lt_last = lt_ref[...]
    p = _softmax_e(lt_last)
    idx, probs = _top2(lt_last, p)
    idx_ref[...] = idx
    probs_ref[...] = probs
    psums = jnp.sum(psum_ref[...], axis=0) + jnp.sum(p, axis=1).reshape(
        1, NUM_EXPERTS)
    mean = psums / jnp.float32(total)
    aux_ref[...] = jnp.sum(mean * jnp.log(mean * NUM_EXPERTS + 1e-9)
                           ).reshape(1, 1)


def kernel(x, W1, b1, W2, b2):
    B, S, H = x.shape
    M = B * S
    x2 = x.reshape(M, H)
    w2t = W2.T.astype(jnp.bfloat16)
    b1r = b1.reshape(1, H)
    b2t = b2.reshape(NUM_EXPERTS, 1)
    nsteps = M // TM

    prev = lambda i: (jnp.maximum(i - 1, 0), 0)

    idx0, probs0, psums, lt_last = pl.pallas_call(
        _router_kernel,
        grid=(nsteps,),
        in_specs=[
            pl.BlockSpec((TM, H), lambda i: (i, 0)),
            pl.BlockSpec((H, H), lambda i: (0, 0)),
            pl.BlockSpec((1, H), lambda i: (0, 0)),
            pl.BlockSpec((NUM_EXPERTS, H), lambda i: (0, 0)),
            pl.BlockSpec((NUM_EXPERTS, 1), lambda i: (0, 0)),
        ],
        out_specs=[
            pl.BlockSpec((TM, TOP_K), prev),
            pl.BlockSpec((TM, TOP_K), prev),
            pl.BlockSpec((1, 1, NUM_EXPERTS), lambda i: (*prev(i), 0)),
            pl.BlockSpec((NUM_EXPERTS, TM), lambda i: (0, 0)),
        ],
        out_shape=[
            jax.ShapeDtypeStruct((M, TOP_K), jnp.int32),
            jax.ShapeDtypeStruct((M, TOP_K), jnp.float32),
            jax.ShapeDtypeStruct((nsteps - 1, 1, NUM_EXPERTS), jnp.float32),
            jax.ShapeDtypeStruct((NUM_EXPERTS, TM), jnp.float32),
        ],
        scratch_shapes=[pltpu.VMEM((HIDDEN, HIDDEN), jnp.bfloat16)],
        compiler_params=pltpu.CompilerParams(
            dimension_semantics=("arbitrary",),
        ),
    )(x2, W1, b1r, w2t, b2t)

    last = nsteps - 1
    idx, probs, aux = pl.pallas_call(
        functools.partial(_final_kernel, total=M),
        grid=(1,),
        in_specs=[
            pl.BlockSpec((nsteps - 1, NUM_EXPERTS), lambda i: (0, 0)),
            pl.BlockSpec((NUM_EXPERTS, TM), lambda i: (0, 0)),
            pl.BlockSpec((TM, TOP_K), lambda i: (last, 0)),
            pl.BlockSpec((TM, TOP_K), lambda i: (last, 0)),
        ],
        out_specs=[
            pl.BlockSpec((TM, TOP_K), lambda i: (last, 0)),
            pl.BlockSpec((TM, TOP_K), lambda i: (last, 0)),
            pl.BlockSpec((1, 1), lambda i: (0, 0)),
        ],
        out_shape=[
            jax.ShapeDtypeStruct((M, TOP_K), jnp.int32),
            jax.ShapeDtypeStruct((M, TOP_K), jnp.float32),
            jax.ShapeDtypeStruct((1, 1), jnp.float32),
        ],
        input_output_aliases={2: 0, 3: 1},
    )(psums.reshape(nsteps - 1, NUM_EXPERTS), lt_last, idx0, probs0)

    return (idx.reshape(B, S, TOP_K), probs.reshape(B, S, TOP_K),
            aux.reshape(()))
